# baseline (speedup 1.0000x reference)
import functools

import jax
import jax.numpy as jnp
from jax.experimental import pallas as pl
from jax.experimental.pallas import tpu as pltpu


def _gcn_kernel(adj_hbm, x_ref, w_ref, b_ref, out_ref, buf, h_ref, x16_ref,
                sems, *, bm, nchunk, slots):
    p = pl.program_id(0)
    i = pl.program_id(1)
    nsteps = pl.num_programs(1)
    g = p * nsteps + i
    ck = bm // nchunk

    def issue(step, slot):
        base = (step % nsteps) * bm
        for c in range(nchunk):
            pltpu.make_async_copy(
                adj_hbm.at[pl.ds(base + c * ck, ck), :],
                buf.at[slot, pl.ds(c * ck, ck), :],
                sems.at[slot],
            ).start()

    @pl.when(g == 0)
    def _():
        x16_ref[...] = x_ref[...].astype(jnp.bfloat16)
        for s in range(slots - 1):
            issue(s, s)

    @pl.when(g + slots - 1 < 2 * nsteps)
    def _():
        issue(g + slots - 1, (g + slots - 1) % slots)

    slot = g % slots
    for c in range(nchunk):
        pltpu.make_async_copy(
            adj_hbm.at[pl.ds(c * ck, ck), :],
            buf.at[slot, pl.ds(c * ck, ck), :],
            sems.at[slot],
        ).wait()

    a16 = buf[slot].astype(jnp.bfloat16)

    @pl.when(p == 0)
    def _():
        t = jnp.dot(a16, x16_ref[...], preferred_element_type=jnp.float32)
        t = jnp.dot(t.astype(jnp.bfloat16), w_ref[0].astype(jnp.bfloat16),
                    preferred_element_type=jnp.float32) + b_ref[0]
        h_ref[pl.ds(i * bm, bm), :] = jnp.maximum(t, 0.0).astype(jnp.bfloat16)

    @pl.when(p == 1)
    def _():
        t = jnp.dot(a16, h_ref[...], preferred_element_type=jnp.float32)
        t = jnp.dot(t.astype(jnp.bfloat16), w_ref[1].astype(jnp.bfloat16),
                    preferred_element_type=jnp.float32) + b_ref[1]
        out_ref[...] = t


def _gcn(x, adj, W1, b1, W2, b2, *, bm, nchunk, slots):
    n, k = adj.shape
    d = W1.shape[1]
    w = jnp.stack([W1, W2])
    b = jnp.stack([b1, b2]).reshape(2, 1, d)
    return pl.pallas_call(
        functools.partial(_gcn_kernel, bm=bm, nchunk=nchunk, slots=slots),
        grid=(2, n // bm),
        in_specs=[
            pl.BlockSpec(memory_space=pl.ANY),
            pl.BlockSpec((k, d), lambda p, i: (0, 0)),
            pl.BlockSpec((2, d, d), lambda p, i: (0, 0, 0)),
            pl.BlockSpec((2, 1, d), lambda p, i: (0, 0, 0)),
        ],
        out_specs=pl.BlockSpec((bm, d), lambda p, i: (p * i, 0)),
        out_shape=jax.ShapeDtypeStruct((n, d), jnp.float32),
        scratch_shapes=[
            pltpu.VMEM((slots, bm, k), jnp.float32),
            pltpu.VMEM((n, d), jnp.bfloat16),
            pltpu.VMEM((k, d), jnp.bfloat16),
            pltpu.SemaphoreType.DMA((slots,)),
        ],
    )(adj, x, w, b)


def kernel(x, adj, W1, b1, W2, b2):
    return _gcn(x, adj, W1, b1, W2, b2, bm=80, nchunk=2, slots=6)


# speedup vs baseline: 1.4301x; 1.4301x over previous
import functools

import jax
import jax.numpy as jnp
from jax.experimental import pallas as pl
from jax.experimental.pallas import tpu as pltpu


def _gcn_kernel(adj_hbm, x_ref, w1_ref, b1_ref, w2_ref, b2_ref, out_ref,
                buf, h_ref, x16_ref, sems, *, bm, nchunk, slots):
    p = pl.program_id(0)
    i = pl.program_id(1)
    nsteps = pl.num_programs(1)
    g = p * nsteps + i
    ck = bm // nchunk

    def issue(step, slot):
        base = (step % nsteps) * bm
        for c in range(nchunk):
            pltpu.make_async_copy(
                adj_hbm.at[pl.ds(base + c * ck, ck), :],
                buf.at[slot, pl.ds(c * ck, ck), :],
                sems.at[slot],
            ).start()

    @pl.when(g == 0)
    def _():
        x16_ref[...] = x_ref[...].astype(jnp.bfloat16)
        for s in range(slots - 1):
            issue(s, s)

    @pl.when(g + slots - 1 < 2 * nsteps)
    def _():
        issue(g + slots - 1, (g + slots - 1) % slots)

    slot = g % slots
    for c in range(nchunk):
        pltpu.make_async_copy(
            adj_hbm.at[pl.ds(c * ck, ck), :],
            buf.at[slot, pl.ds(c * ck, ck), :],
            sems.at[slot],
        ).wait()

    a16 = buf[slot].astype(jnp.bfloat16)

    @pl.when(p == 0)
    def _():
        t = jnp.dot(a16, x16_ref[...], preferred_element_type=jnp.float32)
        t = jnp.dot(t.astype(jnp.bfloat16), w1_ref[...].astype(jnp.bfloat16),
                    preferred_element_type=jnp.float32) + b1_ref[...]
        h_ref[pl.ds(i * bm, bm), :] = jnp.maximum(t, 0.0).astype(jnp.bfloat16)

    @pl.when(p == 1)
    def _():
        t = jnp.dot(a16, h_ref[...], preferred_element_type=jnp.float32)
        t = jnp.dot(t.astype(jnp.bfloat16), w2_ref[...].astype(jnp.bfloat16),
                    preferred_element_type=jnp.float32) + b2_ref[...]
        out_ref[...] = t


def _gcn(x, adj, W1, b1, W2, b2, *, bm, nchunk, slots):
    n, k = adj.shape
    d = W1.shape[1]
    return pl.pallas_call(
        functools.partial(_gcn_kernel, bm=bm, nchunk=nchunk, slots=slots),
        grid=(2, n // bm),
        in_specs=[
            pl.BlockSpec(memory_space=pl.ANY),
            pl.BlockSpec((k, d), lambda p, i: (0, 0)),
            pl.BlockSpec((d, d), lambda p, i: (0, 0)),
            pl.BlockSpec((1, d), lambda p, i: (0, 0)),
            pl.BlockSpec((d, d), lambda p, i: (0, 0)),
            pl.BlockSpec((1, d), lambda p, i: (0, 0)),
        ],
        out_specs=pl.BlockSpec((bm, d), lambda p, i: (p * i, 0)),
        out_shape=jax.ShapeDtypeStruct((n, d), jnp.float32),
        compiler_params=pltpu.CompilerParams(
            vmem_limit_bytes=63 * 1024 * 1024),
        scratch_shapes=[
            pltpu.VMEM((slots, bm, k), jnp.float32),
            pltpu.VMEM((n, d), jnp.bfloat16),
            pltpu.VMEM((k, d), jnp.bfloat16),
            pltpu.SemaphoreType.DMA((slots,)),
        ],
    )(adj, x, W1, b1.reshape(1, d), W2, b2.reshape(1, d))


def kernel(x, adj, W1, b1, W2, b2):
    return _gcn(x, adj, W1, b1, W2, b2, bm=400, nchunk=5, slots=2)


# speedup vs baseline: 1.4345x; 1.0030x over previous
import functools

import jax
import jax.numpy as jnp
from jax.experimental import pallas as pl
from jax.experimental.pallas import tpu as pltpu


def _gcn_kernel(adj_hbm, x_ref, w1_ref, b1_ref, w2_ref, b2_ref, out_ref,
                buf, h_ref, x16_ref, sems, *, bm, nchunk, slots):
    p = pl.program_id(0)
    i = pl.program_id(1)
    nsteps = pl.num_programs(1)
    g = p * nsteps + i
    ck = bm // nchunk

    def issue(step, slot):
        base = (step % nsteps) * bm
        for c in range(nchunk):
            pltpu.make_async_copy(
                adj_hbm.at[pl.ds(base + c * ck, ck), :],
                buf.at[slot, pl.ds(c * ck, ck), :],
                sems.at[slot],
            ).start()

    @pl.when(g == 0)
    def _():
        x16_ref[...] = x_ref[...].astype(jnp.bfloat16)
        for s in range(slots - 1):
            issue(s, s)

    @pl.when(g + slots - 1 < 2 * nsteps)
    def _():
        issue(g + slots - 1, (g + slots - 1) % slots)

    slot = g % slots
    for c in range(nchunk):
        pltpu.make_async_copy(
            adj_hbm.at[pl.ds(c * ck, ck), :],
            buf.at[slot, pl.ds(c * ck, ck), :],
            sems.at[slot],
        ).wait()

    a16 = buf[slot].astype(jnp.bfloat16)

    @pl.when(p == 0)
    def _():
        t = jnp.dot(a16, x16_ref[...], preferred_element_type=jnp.float32)
        t = jnp.dot(t.astype(jnp.bfloat16), w1_ref[...].astype(jnp.bfloat16),
                    preferred_element_type=jnp.float32) + b1_ref[...]
        h_ref[pl.ds(i * bm, bm), :] = jnp.maximum(t, 0.0).astype(jnp.bfloat16)

    @pl.when(p == 1)
    def _():
        t = jnp.dot(a16, h_ref[...], preferred_element_type=jnp.float32)
        t = jnp.dot(t.astype(jnp.bfloat16), w2_ref[...].astype(jnp.bfloat16),
                    preferred_element_type=jnp.float32) + b2_ref[...]
        out_ref[...] = t


def _gcn(x, adj, W1, b1, W2, b2, *, bm, nchunk, slots):
    n, k = adj.shape
    d = W1.shape[1]
    return pl.pallas_call(
        functools.partial(_gcn_kernel, bm=bm, nchunk=nchunk, slots=slots),
        grid=(2, n // bm),
        in_specs=[
            pl.BlockSpec(memory_space=pl.ANY),
            pl.BlockSpec((k, d), lambda p, i: (0, 0)),
            pl.BlockSpec((d, d), lambda p, i: (0, 0)),
            pl.BlockSpec((1, d), lambda p, i: (0, 0)),
            pl.BlockSpec((d, d), lambda p, i: (0, 0)),
            pl.BlockSpec((1, d), lambda p, i: (0, 0)),
        ],
        out_specs=pl.BlockSpec((bm, d), lambda p, i: (p * i, 0)),
        out_shape=jax.ShapeDtypeStruct((n, d), jnp.float32),
        compiler_params=pltpu.CompilerParams(
            vmem_limit_bytes=63 * 1024 * 1024),
        scratch_shapes=[
            pltpu.VMEM((slots, bm, k), jnp.float32),
            pltpu.VMEM((n, d), jnp.bfloat16),
            pltpu.VMEM((k, d), jnp.bfloat16),
            pltpu.SemaphoreType.DMA((slots,)),
        ],
    )(adj, x, W1, b1.reshape(1, d), W2, b2.reshape(1, d))


def kernel(x, adj, W1, b1, W2, b2):
    return _gcn(x, adj, W1, b1, W2, b2, bm=400, nchunk=1, slots=2)
